# baseline (speedup 1.0000x reference)
import functools

import jax
import jax.numpy as jnp
from jax import lax
from jax.experimental import pallas as pl
from jax.experimental.pallas import tpu as pltpu
from jax.experimental.pallas import tpu_sc as plsc

MAX_SPAN = 255
QUERY_LENGTH = 254
KEY_LENGTH = 256
DEPTH = 512
TABLE_ROWS = MAX_SPAN * 2 - 1

_NUM_CORES = 2
_NUM_SUBCORES = 16
_NUM_WORKERS = _NUM_CORES * _NUM_SUBCORES
_Q_PER_WORKER = -(-QUERY_LENGTH // _NUM_WORKERS)

_CHUNK = 64
_NBUF = 2

_ROT_ROWS = [504, 504, 504, 504, 504, 504, 496, 496]
_PACKED_ROWS = sum(_ROT_ROWS)
_SPMEM_ROT = 4
_SPMEM_ROWS = sum(_ROT_ROWS[:_SPMEM_ROT])


def _pack_rotated_tables(weight):
    wpad = jnp.pad(weight, ((0, 8), (0, 0)))
    parts = [
        lax.slice_in_dim(wpad, r, r + n, axis=0)
        for r, n in enumerate(_ROT_ROWS)
    ]
    return jnp.concatenate(parts, axis=0)


def _src_row(q):
    s = (MAX_SPAN - 2) - q
    r = lax.rem(s, 8)
    a = s - r
    off = r * _ROT_ROWS[0] - jnp.maximum(r - 6, 0) * 8
    return pl.multiple_of(off + a, 8)


def _make_sc_kernel():
    mesh = plsc.VectorSubcoreMesh(core_axis_name="c", subcore_axis_name="s")

    @functools.partial(
        pl.kernel,
        mesh=mesh,
        out_type=jax.ShapeDtypeStruct(
            (QUERY_LENGTH, KEY_LENGTH, DEPTH), jnp.float32
        ),
        scratch_types=[
            pltpu.VMEM_SHARED((_SPMEM_ROWS, DEPTH), jnp.float32),
            [pltpu.VMEM((_CHUNK, DEPTH), jnp.float32) for _ in range(_NBUF)],
            pltpu.SemaphoreType.DMA,
            [pltpu.SemaphoreType.DMA for _ in range(_NBUF)],
            [pltpu.SemaphoreType.DMA for _ in range(_NBUF)],
        ],
    )
    def sc_kernel(w8_hbm, out_hbm, shared, bufs, sem, gsems, ssems):
        cid = lax.axis_index("c")
        sid = lax.axis_index("s")
        wid = sid * _NUM_CORES + cid
        rot = lax.rem(MAX_SPAN - 2 - wid + 64, 8)
        use_spmem = rot < _SPMEM_ROT

        stripe = 128
        tail = _SPMEM_ROWS - (_NUM_SUBCORES - 1) * stripe

        @pl.when(sid < _NUM_SUBCORES - 1)
        def _load():
            pltpu.sync_copy(
                w8_hbm.at[pl.ds(sid * stripe, stripe)],
                shared.at[pl.ds(sid * stripe, stripe)],
            )

        @pl.when(sid == _NUM_SUBCORES - 1)
        def _load_tail():
            base = (_NUM_SUBCORES - 1) * stripe
            pltpu.sync_copy(
                w8_hbm.at[pl.ds(base, tail)],
                shared.at[pl.ds(base, tail)],
            )

        plsc.subcore_barrier()

        spmem_copies = []
        for t in range(_Q_PER_WORKER):
            q = wid + _NUM_WORKERS * t
            qc = jnp.minimum(q, QUERY_LENGTH - 1)
            src = jnp.minimum(_src_row(qc), _SPMEM_ROWS - KEY_LENGTH)
            src = pl.multiple_of(src, 8)
            desc = pltpu.make_async_copy(
                shared.at[pl.ds(src, KEY_LENGTH), :], out_hbm.at[qc], sem
            )
            go = jnp.logical_and(use_spmem, q < QUERY_LENGTH)
            spmem_copies.append((go, desc))

            @pl.when(go)
            def _start(desc=desc):
                desc.start()

        pipeline = []
        for t in range(_Q_PER_WORKER):
            q = wid + _NUM_WORKERS * t
            qc = jnp.minimum(q, QUERY_LENGTH - 1)
            src = _src_row(qc)
            for c in range(KEY_LENGTH // _CHUNK):
                i = len(pipeline)
                slot = i % _NBUF
                gd = pltpu.make_async_copy(
                    w8_hbm.at[pl.ds(src + c * _CHUNK, _CHUNK)],
                    bufs[slot],
                    gsems[slot],
                )
                sd = pltpu.make_async_copy(
                    bufs[slot],
                    out_hbm.at[qc, pl.ds(c * _CHUNK, _CHUNK), :],
                    ssems[slot],
                )
                go = jnp.logical_and(
                    jnp.logical_not(use_spmem), q < QUERY_LENGTH
                )
                pipeline.append((go, gd, sd))

                if i >= _NBUF:
                    prev_go, _, prev_sd = pipeline[i - _NBUF]

                    @pl.when(prev_go)
                    def _free(prev_sd=prev_sd):
                        prev_sd.wait()

                @pl.when(go)
                def _bounce(gd=gd, sd=sd):
                    gd.start()
                    gd.wait()
                    sd.start()

        for go, _, sd in pipeline[-_NBUF:]:

            @pl.when(go)
            def _drain(sd=sd):
                sd.wait()

        for go, desc in spmem_copies:

            @pl.when(go)
            def _wait(desc=desc):
                desc.wait()

    return sc_kernel


def kernel(weight):
    return _make_sc_kernel()(_pack_rotated_tables(weight))


# speedup vs baseline: 1.2106x; 1.2106x over previous
import functools

import jax
import jax.numpy as jnp
from jax import lax
from jax.experimental import pallas as pl
from jax.experimental.pallas import tpu as pltpu
from jax.experimental.pallas import tpu_sc as plsc

MAX_SPAN = 255
QUERY_LENGTH = 254
KEY_LENGTH = 256
DEPTH = 512
TABLE_ROWS = MAX_SPAN * 2 - 1

_NUM_CORES = 2
_NUM_SUBCORES = 16
_NUM_WORKERS = _NUM_CORES * _NUM_SUBCORES
_Q_PER_WORKER = -(-QUERY_LENGTH // _NUM_WORKERS)

_ROT_ROWS = [504, 504, 504, 504, 504, 504, 496, 496]
_PACKED_ROWS = sum(_ROT_ROWS)


def _pack_body(w_ref, o_ref):
    base = 0
    for r, n in enumerate(_ROT_ROWS):
        o_ref[base : base + n] = w_ref[r : r + n]
        base += n


def _pack_rotated_tables(weight):
    return pl.pallas_call(
        _pack_body,
        in_specs=[pl.BlockSpec((TABLE_ROWS, DEPTH), lambda: (0, 0))],
        out_specs=pl.BlockSpec((_PACKED_ROWS, DEPTH), lambda: (0, 0)),
        out_shape=jax.ShapeDtypeStruct((_PACKED_ROWS, DEPTH), jnp.float32),
    )(weight)


def _make_sc_kernel():
    mesh = plsc.VectorSubcoreMesh(core_axis_name="c", subcore_axis_name="s")

    @functools.partial(
        pl.kernel,
        mesh=mesh,
        out_type=jax.ShapeDtypeStruct(
            (QUERY_LENGTH, KEY_LENGTH, DEPTH), jnp.float32
        ),
        scratch_types=[
            pltpu.VMEM_SHARED((_PACKED_ROWS, DEPTH), jnp.float32),
            pltpu.SemaphoreType.DMA,
        ],
    )
    def sc_kernel(w8_hbm, out_hbm, shared, sem):
        cid = lax.axis_index("c")
        sid = lax.axis_index("s")
        wid = sid * _NUM_CORES + cid

        stripe = 256
        tail = _PACKED_ROWS - (_NUM_SUBCORES - 1) * stripe

        @pl.when(sid < _NUM_SUBCORES - 1)
        def _load():
            pltpu.sync_copy(
                w8_hbm.at[pl.ds(sid * stripe, stripe)],
                shared.at[pl.ds(sid * stripe, stripe)],
            )

        @pl.when(sid == _NUM_SUBCORES - 1)
        def _load_tail():
            base = (_NUM_SUBCORES - 1) * stripe
            pltpu.sync_copy(
                w8_hbm.at[pl.ds(base, tail)],
                shared.at[pl.ds(base, tail)],
            )

        plsc.subcore_barrier()

        copies = []
        for t in range(_Q_PER_WORKER):
            q = wid + _NUM_WORKERS * t
            qc = jnp.minimum(q, QUERY_LENGTH - 1)
            s = (MAX_SPAN - 2) - qc
            r = lax.rem(s, 8)
            a = s - r
            off = r * _ROT_ROWS[0] - jnp.maximum(r - 6, 0) * 8
            src = pl.multiple_of(off + a, 8)
            desc = pltpu.make_async_copy(
                shared.at[pl.ds(src, KEY_LENGTH), :], out_hbm.at[qc], sem
            )
            copies.append((q, desc))

            @pl.when(q < QUERY_LENGTH)
            def _start(desc=desc):
                desc.start()

        for q, desc in copies:

            @pl.when(q < QUERY_LENGTH)
            def _wait(desc=desc):
                desc.wait()

    return sc_kernel


def kernel(weight):
    return _make_sc_kernel()(_pack_rotated_tables(weight))


# speedup vs baseline: 1.2657x; 1.0455x over previous
import functools

import jax
import jax.numpy as jnp
from jax import lax
from jax.experimental import pallas as pl
from jax.experimental.pallas import tpu as pltpu
from jax.experimental.pallas import tpu_sc as plsc

MAX_SPAN = 255
QUERY_LENGTH = 254
KEY_LENGTH = 256
DEPTH = 512
TABLE_ROWS = MAX_SPAN * 2 - 1

_NUM_CORES = 2
_NUM_SUBCORES = 16
_NUM_WORKERS = _NUM_CORES * _NUM_SUBCORES
_Q_PER_WORKER = -(-QUERY_LENGTH // _NUM_WORKERS)

_ROT_ROWS = [504, 504, 504, 504, 504, 504, 496, 496]
_PACKED_ROWS = sum(_ROT_ROWS)


def _pack_body(w_ref, o_ref):
    base = 0
    for r, n in enumerate(_ROT_ROWS):
        o_ref[base : base + n] = w_ref[r : r + n]
        base += n


def _pack_rotated_tables(weight):
    return pl.pallas_call(
        _pack_body,
        in_specs=[pl.BlockSpec((TABLE_ROWS, DEPTH), lambda: (0, 0))],
        out_specs=pl.BlockSpec((_PACKED_ROWS, DEPTH), lambda: (0, 0)),
        out_shape=jax.ShapeDtypeStruct((_PACKED_ROWS, DEPTH), jnp.float32),
    )(weight)


def _make_sc_kernel():
    mesh = plsc.VectorSubcoreMesh(core_axis_name="c", subcore_axis_name="s")

    half = _PACKED_ROWS // 2
    sc1_rows = _PACKED_ROWS - half

    @functools.partial(
        pl.kernel,
        mesh=mesh,
        out_type=jax.ShapeDtypeStruct(
            (QUERY_LENGTH, KEY_LENGTH, DEPTH), jnp.float32
        ),
        scratch_types=[
            pltpu.VMEM_SHARED((half, DEPTH), jnp.float32),
            pltpu.SemaphoreType.DMA,
        ],
    )
    def sc_kernel(w8_hbm, out_hbm, shared, sem):
        cid = lax.axis_index("c")
        sid = lax.axis_index("s")

        stripe = 128
        hbase = cid * half

        @pl.when(sid < _NUM_SUBCORES - 1)
        def _load():
            pltpu.sync_copy(
                w8_hbm.at[pl.ds(hbase + sid * stripe, stripe)],
                shared.at[pl.ds(sid * stripe, stripe)],
            )

        tbase = (_NUM_SUBCORES - 1) * stripe

        @pl.when(jnp.logical_and(sid == _NUM_SUBCORES - 1, cid == 0))
        def _load_tail0():
            pltpu.sync_copy(
                w8_hbm.at[pl.ds(tbase, half - tbase)],
                shared.at[pl.ds(tbase, half - tbase)],
            )

        @pl.when(jnp.logical_and(sid == _NUM_SUBCORES - 1, cid == 1))
        def _load_tail1():
            pltpu.sync_copy(
                w8_hbm.at[pl.ds(half + tbase, sc1_rows - tbase)],
                shared.at[pl.ds(tbase, sc1_rows - tbase)],
            )

        plsc.subcore_barrier()

        r_local = lax.rem(sid, 4)
        j = lax.div(sid, 4)
        r = 4 * cid + r_local
        roff = r_local * _ROT_ROWS[0] - jnp.maximum(r - 6, 0) * 8

        copies = []
        for t in range(_Q_PER_WORKER):
            k = j + 4 * t
            q = (MAX_SPAN - 2) - 8 * k - r
            qc = jnp.maximum(q, 0)
            src = jnp.minimum(roff + 8 * k, half - KEY_LENGTH)
            src = pl.multiple_of(src, 8)
            desc = pltpu.make_async_copy(
                shared.at[pl.ds(src, KEY_LENGTH), :], out_hbm.at[qc], sem
            )
            copies.append((q, desc))

            @pl.when(q >= 0)
            def _start(desc=desc):
                desc.start()

        for q, desc in copies:

            @pl.when(q >= 0)
            def _wait(desc=desc):
                desc.wait()

    return sc_kernel


def kernel(weight):
    return _make_sc_kernel()(_pack_rotated_tables(weight))


# speedup vs baseline: 1.2677x; 1.0016x over previous
import functools

import jax
import jax.numpy as jnp
from jax import lax
from jax.experimental import pallas as pl
from jax.experimental.pallas import tpu as pltpu
from jax.experimental.pallas import tpu_sc as plsc

MAX_SPAN = 255
QUERY_LENGTH = 254
KEY_LENGTH = 256
DEPTH = 512
TABLE_ROWS = MAX_SPAN * 2 - 1

_NUM_CORES = 2
_NUM_SUBCORES = 16
_NUM_WORKERS = _NUM_CORES * _NUM_SUBCORES
_Q_PER_WORKER = -(-QUERY_LENGTH // _NUM_WORKERS)

_ROT_ROWS = [504, 504, 504, 504, 504, 504, 496, 496]
_PACKED_ROWS = sum(_ROT_ROWS)


def _pack_body(w_ref, o_ref):
    base = 0
    for r, n in enumerate(_ROT_ROWS):
        o_ref[base : base + n] = w_ref[r : r + n]
        base += n


def _pack_rotated_tables(weight):
    return pl.pallas_call(
        _pack_body,
        in_specs=[pl.BlockSpec((TABLE_ROWS, DEPTH), lambda: (0, 0))],
        out_specs=pl.BlockSpec((_PACKED_ROWS, DEPTH), lambda: (0, 0)),
        out_shape=jax.ShapeDtypeStruct((_PACKED_ROWS, DEPTH), jnp.float32),
    )(weight)


def _make_sc_kernel():
    mesh = plsc.VectorSubcoreMesh(core_axis_name="c", subcore_axis_name="s")

    half = _PACKED_ROWS // 2
    sc1_rows = _PACKED_ROWS - half

    @functools.partial(
        pl.kernel,
        mesh=mesh,
        out_type=jax.ShapeDtypeStruct(
            (QUERY_LENGTH, KEY_LENGTH, DEPTH), jnp.float32
        ),
        scratch_types=[
            pltpu.VMEM_SHARED((half, DEPTH), jnp.float32),
            pltpu.SemaphoreType.DMA,
        ],
    )
    def sc_kernel(w8_hbm, out_hbm, shared, sem):
        cid = lax.axis_index("c")
        sid = lax.axis_index("s")

        stripe = 128
        tbase = (_NUM_SUBCORES - 1) * stripe

        for core, hbase, rows in ((0, 0, half), (1, half, sc1_rows)):

            @pl.when(
                jnp.logical_and(cid == core, sid < _NUM_SUBCORES - 1)
            )
            def _load(hbase=hbase):
                pltpu.sync_copy(
                    w8_hbm.at[pl.ds(hbase + sid * stripe, stripe)],
                    shared.at[pl.ds(sid * stripe, stripe)],
                )

            @pl.when(
                jnp.logical_and(cid == core, sid == _NUM_SUBCORES - 1)
            )
            def _load_tail(hbase=hbase, rows=rows):
                pltpu.sync_copy(
                    w8_hbm.at[pl.ds(hbase + rows - stripe, stripe)],
                    shared.at[pl.ds(rows - stripe, stripe)],
                )

        plsc.subcore_barrier()

        r_local = lax.rem(sid, 4)
        j = lax.div(sid, 4)
        wid = lax.rem(8 * j + r_local + 2 + 4 * cid, 32)

        copies = []
        for t in range(_Q_PER_WORKER):
            q = wid + _NUM_WORKERS * t
            qc = jnp.where(q < QUERY_LENGTH, q, q - 8)
            s = (MAX_SPAN - 2) - qc
            r = lax.rem(s, 8)
            a = s - r
            off = r * _ROT_ROWS[0] - jnp.maximum(r - 6, 0) * 8
            src = pl.multiple_of(off + a - half * cid, 8)
            desc = pltpu.make_async_copy(
                shared.at[pl.ds(src, KEY_LENGTH), :], out_hbm.at[qc], sem
            )
            copies.append((q, desc))

            @pl.when(q < QUERY_LENGTH)
            def _start(desc=desc):
                desc.start()

        for q, desc in copies:

            @pl.when(q < QUERY_LENGTH)
            def _wait(desc=desc):
                desc.wait()

    return sc_kernel


def kernel(weight):
    return _make_sc_kernel()(_pack_rotated_tables(weight))
